# baseline (speedup 1.0000x reference)
import jax
import jax.numpy as jnp
from jax.experimental import pallas as pl

_INTERPRET = False

_RANK_CHUNK = 256


def _scores_body(nf_ref, noise_ref, w_ref, b_ref, s_ref, min_ref):
    x = nf_ref[0] + 0.05 * noise_ref[0]
    s = jnp.sum(x * w_ref[...], axis=1, keepdims=True)
    s = s + b_ref[0, 0]
    s_ref[0] = s
    min_ref[...] = jnp.min(s).reshape(1, 1, 1)


def _bitonic_sort_desc(x):
    r, l = x.shape
    n = r * l
    riota = jax.lax.broadcasted_iota(jnp.int32, (r, l), 0)
    ciota = jax.lax.broadcasted_iota(jnp.int32, (r, l), 1)
    k = 2
    while k <= n:
        asc = ((riota * l + ciota) & k) != 0 if k < n else (ciota < 0)
        d = k // 2
        while d >= 1:
            if d < l:
                bit = (ciota & d) != 0
                vp = jnp.where(bit, jnp.roll(x, d, axis=1),
                               jnp.roll(x, -d, axis=1))
            else:
                e = d // l
                bit = (riota & e) != 0
                vp = jnp.where(bit, jnp.roll(x, e, axis=0),
                               jnp.roll(x, -e, axis=0))
            want_min = asc != bit
            x = jnp.where(want_min, jnp.minimum(x, vp), jnp.maximum(x, vp))
            d //= 2
        k *= 2
    return x


def _prefix_sum_2d(x, riota, ciota):
    r, l = x.shape
    d = 1
    while d < l:
        x = x + jnp.where(ciota >= d, jnp.roll(x, d, axis=1), 0.0)
        d *= 2
    rowtot = jax.lax.slice(x, (0, l - 1), (r, l))
    t = rowtot
    e = 1
    while e < r:
        t = t + jnp.where(jax.lax.slice(riota, (0, 0), (r, 1)) >= e,
                          jnp.roll(t, e, axis=0), 0.0)
        e *= 2
    return x + (t - rowtot)


def _sort_body(s_ref, minv_ref, m_ref,
               u_ref, ru_ref, d_ref, v_ref, rv_ref):
    r, l = s_ref.shape[1], s_ref.shape[2]
    fill = jnp.min(minv_ref[...]) - 1.0
    mb = m_ref[0] != 0
    s = jnp.where(mb, s_ref[0], fill)

    c0 = (jnp.max(s) + jnp.min(s)) * 0.5

    mf = mb.astype(jnp.float32)
    eb = jnp.exp(s - c0)
    u_ref[0] = mf * eb
    ru_ref[0] = mf / eb
    d_ref[0] = 1.0 - mf

    ss = _bitonic_sort_desc(s)

    bv = jnp.exp(ss - c0)
    av = 1.0 / bv
    riota = jax.lax.broadcasted_iota(jnp.int32, (r, l), 0)
    ciota = jax.lax.broadcasted_iota(jnp.int32, (r, l), 1)
    pa = _prefix_sum_2d(av, riota, ciota)
    pb = _prefix_sum_2d(bv, riota, ciota)
    bt = jnp.sum(bv)
    denom = bv * pa + av * (bt - pb)
    rd = 1.0 / denom
    v_ref[0] = bv * rd
    rv_ref[0] = av * rd


def _perm_body(u_ref, ru_ref, d_ref, v_ref, rv_ref, out_ref):
    n, ibk = out_ref.shape[1], out_ref.shape[2]
    ib = pl.program_id(1)
    vrow = v_ref[0]
    rvrow = rv_ref[0]
    nq = n // ibk
    for q in range(nq):
        sl = pl.ds(q * ibk, ibk)

        @pl.when(ib == q)
        def _():
            eye = (jax.lax.broadcasted_iota(jnp.int32, (ibk, ibk), 0) ==
                   jax.lax.broadcasted_iota(jnp.int32, (ibk, ibk), 1))
            p = jnp.minimum(u_ref[0, sl, :] * rvrow, ru_ref[0, sl, :] * vrow)
            out_ref[0, sl, :] = jnp.where(eye, p + d_ref[0, sl, :], p)

        @pl.when(ib != q)
        def _():
            out_ref[0, sl, :] = jnp.minimum(u_ref[0, sl, :] * rvrow,
                                            ru_ref[0, sl, :] * vrow)


def kernel(node_features, mask, W, b, noise):
    B, N, D = node_features.shape
    mask_i = mask.astype(jnp.int32)
    w_row = W.reshape(1, D)
    b2 = b.reshape(1, 1)

    scores_col, minv = pl.pallas_call(
        _scores_body,
        grid=(B,),
        in_specs=[
            pl.BlockSpec((1, N, D), lambda i: (i, 0, 0)),
            pl.BlockSpec((1, N, D), lambda i: (i, 0, 0)),
            pl.BlockSpec((1, D), lambda i: (0, 0)),
            pl.BlockSpec((1, 1), lambda i: (0, 0)),
        ],
        out_specs=[
            pl.BlockSpec((1, N, 1), lambda i: (i, 0, 0)),
            pl.BlockSpec((1, 1, 1), lambda i: (i, 0, 0)),
        ],
        out_shape=[
            jax.ShapeDtypeStruct((B, N, 1), jnp.float32),
            jax.ShapeDtypeStruct((B, 1, 1), jnp.float32),
        ],
        interpret=_INTERPRET,
    )(node_features, noise, w_row, b2)

    R, L = N // 128, 128
    s16 = scores_col.reshape(B, R, L)
    m16 = mask_i.reshape(B, R, L)

    f16 = pl.BlockSpec((1, R, L), lambda i: (i, 0, 0))
    o16 = jax.ShapeDtypeStruct((B, R, L), jnp.float32)
    u16, ru16, d16, v16, rv16 = pl.pallas_call(
        _sort_body,
        grid=(B,),
        in_specs=[
            f16,
            pl.BlockSpec((B, 1, 1), lambda i: (0, 0, 0)),
            f16,
        ],
        out_specs=[f16, f16, f16, f16, f16],
        out_shape=[o16, o16, o16, o16, o16],
        interpret=_INTERPRET,
    )(s16, minv, m16)

    ucol = u16.reshape(B, N, 1)
    rucol = ru16.reshape(B, N, 1)
    dcol = d16.reshape(B, N, 1)
    vrow = v16.reshape(B, 1, N)
    rvrow = rv16.reshape(B, 1, N)

    IBK = 512
    out = pl.pallas_call(
        _perm_body,
        grid=(B, N // IBK),
        in_specs=[
            pl.BlockSpec((1, N, 1), lambda bb, ib: (bb, 0, 0)),
            pl.BlockSpec((1, N, 1), lambda bb, ib: (bb, 0, 0)),
            pl.BlockSpec((1, N, 1), lambda bb, ib: (bb, 0, 0)),
            pl.BlockSpec((1, 1, IBK), lambda bb, ib: (bb, 0, ib)),
            pl.BlockSpec((1, 1, IBK), lambda bb, ib: (bb, 0, ib)),
        ],
        out_specs=pl.BlockSpec((1, N, IBK), lambda bb, ib: (bb, 0, ib)),
        out_shape=jax.ShapeDtypeStruct((B, N, N), jnp.float32),
        interpret=_INTERPRET,
    )(ucol, rucol, dcol, vrow, rvrow)
    return out


# speedup vs baseline: 1.1992x; 1.1992x over previous
import jax
import jax.numpy as jnp
from jax.experimental import pallas as pl

_INTERPRET = False

_RANK_CHUNK = 256


def _scores_body(nf_ref, noise_ref, w_ref, b_ref, s_ref, min_ref):
    x = nf_ref[0] + 0.05 * noise_ref[0]
    s = jnp.sum(x * w_ref[...], axis=1, keepdims=True)
    s = s + b_ref[0, 0]
    s_ref[0] = s
    min_ref[...] = jnp.min(s).reshape(1, 1, 1)


def _bitonic_sort_desc(x):
    _, r, l = x.shape
    n = r * l
    riota = jax.lax.broadcasted_iota(jnp.int32, x.shape, 1)
    ciota = jax.lax.broadcasted_iota(jnp.int32, x.shape, 2)
    k = 2
    while k <= n:
        asc = ((riota * l + ciota) & k) != 0 if k < n else (ciota < 0)
        d = k // 2
        while d >= 1:
            if d < l:
                bit = (ciota & d) != 0
                vp = jnp.where(bit, jnp.roll(x, d, axis=2),
                               jnp.roll(x, -d, axis=2))
            else:
                e = d // l
                bit = (riota & e) != 0
                vp = jnp.where(bit, jnp.roll(x, e, axis=1),
                               jnp.roll(x, -e, axis=1))
            want_min = asc != bit
            x = jnp.where(want_min, jnp.minimum(x, vp), jnp.maximum(x, vp))
            d //= 2
        k *= 2
    return x


def _prefix_sum_2d(x, riota, ciota):
    bb, r, l = x.shape
    d = 1
    while d < l:
        x = x + jnp.where(ciota >= d, jnp.roll(x, d, axis=2), 0.0)
        d *= 2
    rowtot = jax.lax.slice(x, (0, 0, l - 1), (bb, r, l))
    t = rowtot
    e = 1
    while e < r:
        t = t + jnp.where(jax.lax.slice(riota, (0, 0, 0), (bb, r, 1)) >= e,
                          jnp.roll(t, e, axis=1), 0.0)
        e *= 2
    return x + (t - rowtot)


def _sort_body(s_ref, minv_ref, m_ref,
               u_ref, ru_ref, d_ref, v_ref, rv_ref):
    fill = jnp.min(minv_ref[...]) - 1.0
    mb = m_ref[...] != 0
    s = jnp.where(mb, s_ref[...], fill)

    c0 = (jnp.max(s, axis=(1, 2), keepdims=True) +
          jnp.min(s, axis=(1, 2), keepdims=True)) * 0.5

    mf = mb.astype(jnp.float32)
    eb = jnp.exp(s - c0)
    u_ref[...] = mf * eb
    ru_ref[...] = mf / eb
    d_ref[...] = 1.0 - mf

    ss = _bitonic_sort_desc(s)

    bv = jnp.exp(ss - c0)
    av = 1.0 / bv
    riota = jax.lax.broadcasted_iota(jnp.int32, ss.shape, 1)
    ciota = jax.lax.broadcasted_iota(jnp.int32, ss.shape, 2)
    pa = _prefix_sum_2d(av, riota, ciota)
    pb = _prefix_sum_2d(bv, riota, ciota)
    bt = jnp.sum(bv, axis=(1, 2), keepdims=True)
    denom = bv * pa + av * (bt - pb)
    rd = 1.0 / denom
    v_ref[...] = bv * rd
    rv_ref[...] = av * rd


def _perm_body(u_ref, ru_ref, d_ref, v_ref, rv_ref, out_ref):
    n, ibk = out_ref.shape[1], out_ref.shape[2]
    ib = pl.program_id(1)
    vrow = v_ref[0]
    rvrow = rv_ref[0]
    nq = n // ibk
    for q in range(nq):
        sl = pl.ds(q * ibk, ibk)

        @pl.when(ib == q)
        def _():
            eye = (jax.lax.broadcasted_iota(jnp.int32, (ibk, ibk), 0) ==
                   jax.lax.broadcasted_iota(jnp.int32, (ibk, ibk), 1))
            p = jnp.minimum(u_ref[0, sl, :] * rvrow, ru_ref[0, sl, :] * vrow)
            out_ref[0, sl, :] = jnp.where(eye, p + d_ref[0, sl, :], p)

        @pl.when(ib != q)
        def _():
            out_ref[0, sl, :] = jnp.minimum(u_ref[0, sl, :] * rvrow,
                                            ru_ref[0, sl, :] * vrow)


def kernel(node_features, mask, W, b, noise):
    B, N, D = node_features.shape
    mask_i = mask.astype(jnp.int32)
    w_row = W.reshape(1, D)
    b2 = b.reshape(1, 1)

    scores_col, minv = pl.pallas_call(
        _scores_body,
        grid=(B,),
        in_specs=[
            pl.BlockSpec((1, N, D), lambda i: (i, 0, 0)),
            pl.BlockSpec((1, N, D), lambda i: (i, 0, 0)),
            pl.BlockSpec((1, D), lambda i: (0, 0)),
            pl.BlockSpec((1, 1), lambda i: (0, 0)),
        ],
        out_specs=[
            pl.BlockSpec((1, N, 1), lambda i: (i, 0, 0)),
            pl.BlockSpec((1, 1, 1), lambda i: (i, 0, 0)),
        ],
        out_shape=[
            jax.ShapeDtypeStruct((B, N, 1), jnp.float32),
            jax.ShapeDtypeStruct((B, 1, 1), jnp.float32),
        ],
        interpret=_INTERPRET,
    )(node_features, noise, w_row, b2)

    R, L = N // 128, 128
    s16 = scores_col.reshape(B, R, L)
    m16 = mask_i.reshape(B, R, L)

    f16 = pl.BlockSpec((B, R, L), lambda: (0, 0, 0))
    o16 = jax.ShapeDtypeStruct((B, R, L), jnp.float32)
    u16, ru16, d16, v16, rv16 = pl.pallas_call(
        _sort_body,
        in_specs=[
            f16,
            pl.BlockSpec((B, 1, 1), lambda: (0, 0, 0)),
            f16,
        ],
        out_specs=[f16, f16, f16, f16, f16],
        out_shape=[o16, o16, o16, o16, o16],
        interpret=_INTERPRET,
    )(s16, minv, m16)

    ucol = u16.reshape(B, N, 1)
    rucol = ru16.reshape(B, N, 1)
    dcol = d16.reshape(B, N, 1)
    vrow = v16.reshape(B, 1, N)
    rvrow = rv16.reshape(B, 1, N)

    IBK = 512
    out = pl.pallas_call(
        _perm_body,
        grid=(B, N // IBK),
        in_specs=[
            pl.BlockSpec((1, N, 1), lambda bb, ib: (bb, 0, 0)),
            pl.BlockSpec((1, N, 1), lambda bb, ib: (bb, 0, 0)),
            pl.BlockSpec((1, N, 1), lambda bb, ib: (bb, 0, 0)),
            pl.BlockSpec((1, 1, IBK), lambda bb, ib: (bb, 0, ib)),
            pl.BlockSpec((1, 1, IBK), lambda bb, ib: (bb, 0, ib)),
        ],
        out_specs=pl.BlockSpec((1, N, IBK), lambda bb, ib: (bb, 0, ib)),
        out_shape=jax.ShapeDtypeStruct((B, N, N), jnp.float32),
        interpret=_INTERPRET,
    )(ucol, rucol, dcol, vrow, rvrow)
    return out


# speedup vs baseline: 1.4628x; 1.2198x over previous
import jax
import jax.numpy as jnp
from jax.experimental import pallas as pl

_INTERPRET = False


def _scores_body(nf_ref, noise_ref, w_ref, b_ref, m_ref,
                 s16_ref, mf16_ref, minv_ref, c0_ref, u_ref, ru_ref, d_ref):
    n, dd = nf_ref.shape[1], nf_ref.shape[2]
    l = 128
    x = nf_ref[0] + 0.05 * noise_ref[0]
    w = w_ref[...]
    bias = b_ref[0, 0]

    scol = jax.lax.dot_general(
        x, w, (((1,), (0,)), ((), ())),
        preferred_element_type=jnp.float32) + bias

    rows = []
    for r in range(n // l):
        xr = jax.lax.slice(x, (r * l, 0), ((r + 1) * l, dd))
        rows.append(jax.lax.dot_general(
            w, xr, (((0,), (1,)), ((), ())),
            preferred_element_type=jnp.float32))
    s16 = jnp.concatenate(rows, axis=0) + bias
    s16_ref[0] = s16

    mf = (m_ref[0] != 0).astype(jnp.float32)
    eye128 = (jax.lax.broadcasted_iota(jnp.int32, (l, l), 0) ==
              jax.lax.broadcasted_iota(jnp.int32, (l, l), 1)
              ).astype(jnp.float32)
    mrows = []
    for r in range(n // l):
        mr = jax.lax.slice(mf, (r * l, 0), ((r + 1) * l, 1))
        mrows.append(jax.lax.dot_general(
            mr, eye128, (((0,), (0,)), ((), ())),
            preferred_element_type=jnp.float32))
    mf16_ref[0] = jnp.concatenate(mrows, axis=0)

    minv_ref[...] = jnp.min(s16).reshape(1, 1, 1)
    c0 = (jnp.max(s16) + jnp.min(s16)) * 0.5
    c0_ref[...] = c0.reshape(1, 1, 1)

    eb = jnp.exp(scol - c0)
    u_ref[0] = mf * eb
    ru_ref[0] = mf / eb
    d_ref[0] = 1.0 - mf


def _bitonic_sort_desc(x):
    _, r, l = x.shape
    n = r * l
    riota = jax.lax.broadcasted_iota(jnp.int32, x.shape, 1)
    ciota = jax.lax.broadcasted_iota(jnp.int32, x.shape, 2)
    k = 2
    while k <= n:
        asc = ((riota * l + ciota) & k) != 0 if k < n else (ciota < 0)
        d = k // 2
        while d >= 1:
            if d < l:
                bit = (ciota & d) != 0
                vp = jnp.where(bit, jnp.roll(x, d, axis=2),
                               jnp.roll(x, -d, axis=2))
            else:
                e = d // l
                bit = (riota & e) != 0
                vp = jnp.where(bit, jnp.roll(x, e, axis=1),
                               jnp.roll(x, -e, axis=1))
            want_min = asc != bit
            x = jnp.where(want_min, jnp.minimum(x, vp), jnp.maximum(x, vp))
            d //= 2
        k *= 2
    return x


def _prefix_sum_2d(x, riota, ciota):
    bb, r, l = x.shape
    d = 1
    while d < l:
        x = x + jnp.where(ciota >= d, jnp.roll(x, d, axis=2), 0.0)
        d *= 2
    rowtot = jax.lax.slice(x, (0, 0, l - 1), (bb, r, l))
    t = rowtot
    e = 1
    while e < r:
        t = t + jnp.where(jax.lax.slice(riota, (0, 0, 0), (bb, r, 1)) >= e,
                          jnp.roll(t, e, axis=1), 0.0)
        e *= 2
    return x + (t - rowtot)


def _sort_body(s16_ref, mf16_ref, minv_ref, c0_ref, v_ref, rv_ref):
    fill = jnp.min(minv_ref[...]) - 1.0
    s = jnp.where(mf16_ref[...] > 0.5, s16_ref[...], fill)
    c0 = c0_ref[...]

    ss = _bitonic_sort_desc(s)

    bv = jnp.exp(ss - c0)
    av = 1.0 / bv
    riota = jax.lax.broadcasted_iota(jnp.int32, ss.shape, 1)
    ciota = jax.lax.broadcasted_iota(jnp.int32, ss.shape, 2)
    pa = _prefix_sum_2d(av, riota, ciota)
    pb = _prefix_sum_2d(bv, riota, ciota)
    bt = jnp.sum(bv, axis=(1, 2), keepdims=True)
    denom = bv * pa + av * (bt - pb)
    rd = 1.0 / denom
    v_ref[...] = bv * rd
    rv_ref[...] = av * rd


def _perm_body(u_ref, ru_ref, d_ref, v_ref, rv_ref, out_ref):
    n, ibk = out_ref.shape[1], out_ref.shape[2]
    ib = pl.program_id(1)
    v4 = v_ref[0, 0]
    rv4 = rv_ref[0, 0]
    nt = ibk // 128
    vrow = jnp.concatenate(
        [jax.lax.slice(v4, (t, 0), (t + 1, 128)) for t in range(nt)], axis=1)
    rvrow = jnp.concatenate(
        [jax.lax.slice(rv4, (t, 0), (t + 1, 128)) for t in range(nt)], axis=1)
    nq = n // ibk
    for q in range(nq):
        sl = pl.ds(q * ibk, ibk)

        @pl.when(ib == q)
        def _():
            eye = (jax.lax.broadcasted_iota(jnp.int32, (ibk, ibk), 0) ==
                   jax.lax.broadcasted_iota(jnp.int32, (ibk, ibk), 1))
            p = jnp.minimum(u_ref[0, sl, :] * rvrow, ru_ref[0, sl, :] * vrow)
            out_ref[0, sl, :] = jnp.where(eye, p + d_ref[0, sl, :], p)

        @pl.when(ib != q)
        def _():
            out_ref[0, sl, :] = jnp.minimum(u_ref[0, sl, :] * rvrow,
                                            ru_ref[0, sl, :] * vrow)


def kernel(node_features, mask, W, b, noise):
    B, N, D = node_features.shape
    R, L = N // 128, 128
    mask_col = mask.astype(jnp.int32).reshape(B, N, 1)
    b2 = b.reshape(1, 1)

    fcol = pl.BlockSpec((1, N, 1), lambda i: (i, 0, 0))
    f16 = pl.BlockSpec((1, R, L), lambda i: (i, 0, 0))
    fsc = pl.BlockSpec((1, 1, 1), lambda i: (i, 0, 0))
    ocol = jax.ShapeDtypeStruct((B, N, 1), jnp.float32)
    o16 = jax.ShapeDtypeStruct((B, R, L), jnp.float32)
    osc = jax.ShapeDtypeStruct((B, 1, 1), jnp.float32)

    s16, mf16, minv, c0v, ucol, rucol, dcol = pl.pallas_call(
        _scores_body,
        grid=(B,),
        in_specs=[
            pl.BlockSpec((1, N, D), lambda i: (i, 0, 0)),
            pl.BlockSpec((1, N, D), lambda i: (i, 0, 0)),
            pl.BlockSpec((D, 1), lambda i: (0, 0)),
            pl.BlockSpec((1, 1), lambda i: (0, 0)),
            fcol,
        ],
        out_specs=[f16, f16, fsc, fsc, fcol, fcol, fcol],
        out_shape=[o16, o16, osc, osc, ocol, ocol, ocol],
        interpret=_INTERPRET,
    )(node_features, noise, W, b2, mask_col)

    fall = pl.BlockSpec((B, R, L), lambda: (0, 0, 0))
    v16, rv16 = pl.pallas_call(
        _sort_body,
        in_specs=[
            fall,
            fall,
            pl.BlockSpec((B, 1, 1), lambda: (0, 0, 0)),
            pl.BlockSpec((B, 1, 1), lambda: (0, 0, 0)),
        ],
        out_specs=[fall, fall],
        out_shape=[o16, o16],
        interpret=_INTERPRET,
    )(s16, mf16, minv, c0v)

    IBK = 512
    NIB = N // IBK
    v16b = v16.reshape(B, NIB, IBK // L, L)
    rv16b = rv16.reshape(B, NIB, IBK // L, L)
    out = pl.pallas_call(
        _perm_body,
        grid=(B, NIB),
        in_specs=[
            pl.BlockSpec((1, N, 1), lambda bb, ib: (bb, 0, 0)),
            pl.BlockSpec((1, N, 1), lambda bb, ib: (bb, 0, 0)),
            pl.BlockSpec((1, N, 1), lambda bb, ib: (bb, 0, 0)),
            pl.BlockSpec((1, 1, IBK // L, L), lambda bb, ib: (bb, ib, 0, 0)),
            pl.BlockSpec((1, 1, IBK // L, L), lambda bb, ib: (bb, ib, 0, 0)),
        ],
        out_specs=pl.BlockSpec((1, N, IBK), lambda bb, ib: (bb, 0, ib)),
        out_shape=jax.ShapeDtypeStruct((B, N, N), jnp.float32),
        interpret=_INTERPRET,
    )(ucol, rucol, dcol, v16b, rv16b)
    return out


# speedup vs baseline: 1.4878x; 1.0171x over previous
import jax
import jax.numpy as jnp
from jax.experimental import pallas as pl
from jax.experimental.pallas import tpu as pltpu

_INTERPRET = False

_SENT = -3.0e38


def _scores_body(nf_ref, noise_ref, w_ref, b_ref, m_ref,
                 s16_ref, minv_ref, c0_ref, u_ref, ru_ref, d_ref):
    n = nf_ref.shape[1]
    l = 128
    x = nf_ref[0] + 0.05 * noise_ref[0]
    scol = jax.lax.dot_general(
        x, w_ref[...], (((1,), (0,)), ((), ())),
        preferred_element_type=jnp.float32) + b_ref[0, 0]

    mn = jnp.min(scol)
    c0 = (jnp.max(scol) + mn) * 0.5
    minv_ref[...] = mn.reshape(1, 1, 1)
    c0_ref[...] = c0.reshape(1, 1, 1)

    mf = (m_ref[0] != 0).astype(jnp.float32)
    eb = jnp.exp(scol - c0)
    u_ref[0] = mf * eb
    ru_ref[0] = mf / eb
    d_ref[0] = 1.0 - mf

    smcol = jnp.where(mf > 0.5, scol, _SENT)
    eye128 = (jax.lax.broadcasted_iota(jnp.int32, (l, l), 0) ==
              jax.lax.broadcasted_iota(jnp.int32, (l, l), 1)
              ).astype(jnp.float32)
    rows = []
    for r in range(n // l):
        smr = jax.lax.slice(smcol, (r * l, 0), ((r + 1) * l, 1))
        rows.append(jax.lax.dot_general(
            smr, eye128, (((0,), (0,)), ((), ())),
            preferred_element_type=jnp.float32))
    s16_ref[0] = jnp.concatenate(rows, axis=0)


def _bitonic_sort_desc(x):
    _, r, l = x.shape
    n = r * l
    riota = jax.lax.broadcasted_iota(jnp.int32, x.shape, 1)
    ciota = jax.lax.broadcasted_iota(jnp.int32, x.shape, 2)
    k = 2
    while k <= n:
        asc = ((riota * l + ciota) & k) != 0 if k < n else (ciota < 0)
        d = k // 2
        while d >= 1:
            if d < l:
                bit = (ciota & d) != 0
                vp = jnp.where(bit, jnp.roll(x, d, axis=2),
                               jnp.roll(x, -d, axis=2))
            else:
                e = d // l
                bit = (riota & e) != 0
                vp = jnp.where(bit, jnp.roll(x, e, axis=1),
                               jnp.roll(x, -e, axis=1))
            want_min = asc != bit
            x = jnp.where(want_min, jnp.minimum(x, vp), jnp.maximum(x, vp))
            d //= 2
        k *= 2
    return x


def _prefix_sum_2d(x, riota, ciota):
    bb, r, l = x.shape
    d = 1
    while d < l:
        x = x + jnp.where(ciota >= d, jnp.roll(x, d, axis=2), 0.0)
        d *= 2
    rowtot = jax.lax.slice(x, (0, 0, l - 1), (bb, r, l))
    t = rowtot
    e = 1
    while e < r:
        t = t + jnp.where(jax.lax.slice(riota, (0, 0, 0), (bb, r, 1)) >= e,
                          jnp.roll(t, e, axis=1), 0.0)
        e *= 2
    return x + (t - rowtot)


def _make_perm_body(nib, ibk, l):
    def _perm_body(s16_ref, minv_ref, c0_ref, u_ref, ru_ref, d_ref,
                   out_ref, v_scr, rv_scr):
        i = pl.program_id(0)
        n = out_ref.shape[1]
        nq = n // ibk

        @pl.when(i == 0)
        def _():
            fill = jnp.min(minv_ref[...]) - 1.0
            s = jnp.where(s16_ref[...] < -1.0e38, fill, s16_ref[...])
            ss = _bitonic_sort_desc(s)
            c0 = c0_ref[...]
            bv = jnp.exp(ss - c0)
            av = 1.0 / bv
            riota = jax.lax.broadcasted_iota(jnp.int32, ss.shape, 1)
            ciota = jax.lax.broadcasted_iota(jnp.int32, ss.shape, 2)
            pa = _prefix_sum_2d(av, riota, ciota)
            pb = _prefix_sum_2d(bv, riota, ciota)
            bt = jnp.sum(bv, axis=(1, 2), keepdims=True)
            denom = bv * pa + av * (bt - pb)
            rd = 1.0 / denom
            v_scr[...] = bv * rd
            rv_scr[...] = av * rd

        @pl.when(i > 0)
        def _():
            t = i - 1
            b = t // nib
            ib = t % nib
            rpb = ibk // l
            v4 = v_scr[pl.ds(b, 1), pl.ds(ib * rpb, rpb), :][0]
            rv4 = rv_scr[pl.ds(b, 1), pl.ds(ib * rpb, rpb), :][0]
            vrow = jnp.concatenate(
                [jax.lax.slice(v4, (tt, 0), (tt + 1, l)) for tt in range(rpb)],
                axis=1)
            rvrow = jnp.concatenate(
                [jax.lax.slice(rv4, (tt, 0), (tt + 1, l)) for tt in range(rpb)],
                axis=1)
            for q in range(nq):
                sl = pl.ds(q * ibk, ibk)

                @pl.when(ib == q)
                def _():
                    eye = (jax.lax.broadcasted_iota(jnp.int32, (ibk, ibk), 0) ==
                           jax.lax.broadcasted_iota(jnp.int32, (ibk, ibk), 1))
                    p = jnp.minimum(u_ref[0, sl, :] * rvrow,
                                    ru_ref[0, sl, :] * vrow)
                    out_ref[0, sl, :] = jnp.where(eye, p + d_ref[0, sl, :], p)

                @pl.when(ib != q)
                def _():
                    out_ref[0, sl, :] = jnp.minimum(u_ref[0, sl, :] * rvrow,
                                                    ru_ref[0, sl, :] * vrow)
    return _perm_body


def kernel(node_features, mask, W, b, noise):
    B, N, D = node_features.shape
    R, L = N // 128, 128
    mask_col = mask.astype(jnp.int32).reshape(B, N, 1)
    b2 = b.reshape(1, 1)

    fcol = pl.BlockSpec((1, N, 1), lambda i: (i, 0, 0))
    fsc = pl.BlockSpec((1, 1, 1), lambda i: (i, 0, 0))
    ocol = jax.ShapeDtypeStruct((B, N, 1), jnp.float32)
    osc = jax.ShapeDtypeStruct((B, 1, 1), jnp.float32)

    s16, minv, c0v, ucol, rucol, dcol = pl.pallas_call(
        _scores_body,
        grid=(B,),
        in_specs=[
            pl.BlockSpec((1, N, D), lambda i: (i, 0, 0)),
            pl.BlockSpec((1, N, D), lambda i: (i, 0, 0)),
            pl.BlockSpec((D, 1), lambda i: (0, 0)),
            pl.BlockSpec((1, 1), lambda i: (0, 0)),
            fcol,
        ],
        out_specs=[pl.BlockSpec((1, R, L), lambda i: (i, 0, 0)),
                   fsc, fsc, fcol, fcol, fcol],
        out_shape=[jax.ShapeDtypeStruct((B, R, L), jnp.float32),
                   osc, osc, ocol, ocol, ocol],
        interpret=_INTERPRET,
    )(node_features, noise, W, b2, mask_col)

    IBK = 512
    NIB = N // IBK

    def bidx(i):
        return jnp.maximum(i - 1, 0) // NIB

    def ibidx(i):
        return jnp.maximum(i - 1, 0) % NIB

    out = pl.pallas_call(
        _make_perm_body(NIB, IBK, L),
        grid=(1 + B * NIB,),
        in_specs=[
            pl.BlockSpec((B, R, L), lambda i: (0, 0, 0)),
            pl.BlockSpec((B, 1, 1), lambda i: (0, 0, 0)),
            pl.BlockSpec((B, 1, 1), lambda i: (0, 0, 0)),
            pl.BlockSpec((1, N, 1), lambda i: (bidx(i), 0, 0)),
            pl.BlockSpec((1, N, 1), lambda i: (bidx(i), 0, 0)),
            pl.BlockSpec((1, N, 1), lambda i: (bidx(i), 0, 0)),
        ],
        out_specs=pl.BlockSpec((1, N, IBK), lambda i: (bidx(i), 0, ibidx(i))),
        out_shape=jax.ShapeDtypeStruct((B, N, N), jnp.float32),
        scratch_shapes=[
            pltpu.VMEM((B, R, L), jnp.float32),
            pltpu.VMEM((B, R, L), jnp.float32),
        ],
        interpret=_INTERPRET,
    )(s16, minv, c0v, ucol, rucol, dcol)
    return out
